# baseline (speedup 1.0000x reference)
import functools

import jax
import jax.numpy as jnp
from jax.experimental import pallas as pl
from jax.experimental.pallas import tpu as pltpu

_L_BLK = 128
_HALO_BLK = 8


def _cdiv(a, b):
    return -(-a // b)


def _conv_tile(pre_ref, x_ref, post_ref, w_ref, kernel_size, pad, l_blk,
               n_total, c_in, n_tiles, t):
    lead = pad
    trail = kernel_size - 1 - pad
    pre = jnp.where(t > 0, pre_ref[_HALO_BLK - lead:, :, :], 0)
    post = jnp.where(t < n_tiles - 1, post_ref[:trail, :, :], 0)
    xc = jnp.concatenate([pre, x_ref[...], post], axis=0)
    taps = [
        jax.lax.slice_in_dim(xc, k, k + l_blk, axis=0)
        for k in range(kernel_size)
    ]
    xs = jnp.concatenate(taps, axis=2)
    xs = xs.reshape(l_blk * n_total, kernel_size * c_in)
    return jax.lax.dot_general(
        xs, w_ref[...],
        dimension_numbers=(((1,), (0,)), ((), ())),
        preferred_element_type=jnp.float32)


def _fused_kernel(pre_ref, x_ref, post_ref, w_ref, g_ref, b_ref, out_ref,
                  acc_ref, *, kernel_size, pad, l_blk, n_total, c_in, l_out,
                  n_tiles, cnt, eps):
    phase = pl.program_id(0)
    t = pl.program_id(1)
    y = _conv_tile(pre_ref, x_ref, post_ref, w_ref, kernel_size, pad, l_blk,
                   n_total, c_in, n_tiles, t)

    @pl.when(phase == 0)
    def _stats():
        s1 = jnp.sum(y, axis=0, keepdims=True)
        s2 = jnp.sum(y * y, axis=0, keepdims=True)
        n_ragged = n_tiles * l_blk - l_out
        if n_ragged:
            yr = y.reshape(l_blk, n_total, -1)[l_blk - n_ragged:]
            yr = yr.reshape(n_ragged * n_total, -1)
            r1 = jnp.sum(yr, axis=0, keepdims=True)
            r2 = jnp.sum(yr * yr, axis=0, keepdims=True)
            last = (t == n_tiles - 1).astype(jnp.float32)
            s1 = s1 - last * r1
            s2 = s2 - last * r2
        s = jnp.concatenate([s1, s2], axis=0)
        acc_ref[...] = jnp.where(t == 0, s, acc_ref[...] + s)

    @pl.when(phase == 1)
    def _apply():
        st = acc_ref[...]
        inv_cnt = jnp.float32(1.0 / cnt)
        mean = st[0:1, :] * inv_cnt
        var = jnp.maximum(st[1:2, :] * inv_cnt - mean * mean, 0.0)
        scale = g_ref[...] * jax.lax.rsqrt(var + eps)
        shift = b_ref[...] - mean * scale
        z = jnp.maximum(y * scale + shift, 0.0)
        out_ref[...] = z.reshape(l_blk, n_total, -1)


def kernel(x, weight, bias, gamma, beta):
    del bias
    kernel_size = weight.shape[2]
    dilation = 1
    eps = 1e-3

    n, c_in, length = x.shape
    c_out = weight.shape[0]
    pad = (dilation * (kernel_size - 1)) // 2
    halo = dilation * (kernel_size - 1)
    l_out = length + 2 * pad - halo
    assert halo < _HALO_BLK + pad and pad < _HALO_BLK
    assert n % 8 == 0 and length % _L_BLK == 0

    n_tiles = _cdiv(l_out, _L_BLK)
    units = _L_BLK // _HALO_BLK
    total_units = length // _HALO_BLK

    xt = jnp.transpose(x, (2, 0, 1)).astype(jnp.bfloat16)
    w_t = jnp.transpose(weight, (2, 1, 0)).reshape(
        kernel_size * c_in, c_out).astype(jnp.bfloat16)
    g2 = gamma.astype(jnp.float32).reshape(1, c_out)
    b2 = beta.astype(jnp.float32).reshape(1, c_out)

    grid = (2, n_tiles)
    pre_spec = pl.BlockSpec(
        (_HALO_BLK, n, c_in),
        lambda p, t: (jnp.maximum(t * units - 1, 0), 0, 0))
    x_spec = pl.BlockSpec((_L_BLK, n, c_in), lambda p, t: (t, 0, 0))
    post_spec = pl.BlockSpec(
        (_HALO_BLK, n, c_in),
        lambda p, t: (jnp.minimum(t * units + units, total_units - 1), 0, 0))
    w_spec = pl.BlockSpec((kernel_size * c_in, c_out), lambda p, t: (0, 0))
    vmem_limit = 100 * 1024 * 1024

    out_t = pl.pallas_call(
        functools.partial(_fused_kernel, kernel_size=kernel_size, pad=pad,
                          l_blk=_L_BLK, n_total=n, c_in=c_in, l_out=l_out,
                          n_tiles=n_tiles, cnt=float(n * l_out), eps=eps),
        out_shape=jax.ShapeDtypeStruct((l_out, n, c_out), jnp.float32),
        grid=grid,
        in_specs=[pre_spec, x_spec, post_spec, w_spec,
                  pl.BlockSpec((1, c_out), lambda p, t: (0, 0)),
                  pl.BlockSpec((1, c_out), lambda p, t: (0, 0))],
        out_specs=pl.BlockSpec((_L_BLK, n, c_out), lambda p, t: (p * t, 0, 0)),
        scratch_shapes=[pltpu.VMEM((2, c_out), jnp.float32)],
        compiler_params=pltpu.CompilerParams(
            dimension_semantics=("arbitrary", "arbitrary"),
            vmem_limit_bytes=vmem_limit),
    )(xt, xt, xt, w_t, g2, b2)

    return jnp.transpose(out_t, (1, 2, 0))


# speedup vs baseline: 1.1056x; 1.1056x over previous
import functools

import jax
import jax.numpy as jnp
from jax.experimental import pallas as pl
from jax.experimental.pallas import tpu as pltpu

_L_BLK = 128
_L_BLK1 = 256
_HALO_BLK = 8


def _cdiv(a, b):
    return -(-a // b)


def _conv_tile(pre_ref, x_ref, post_ref, w_ref, kernel_size, pad, l_blk,
               n_total, c_in, n_tiles):
    t = pl.program_id(0)
    lead = pad
    trail = kernel_size - 1 - pad
    pre = jnp.where(t > 0, pre_ref[_HALO_BLK - lead:, :, :], 0)
    post = jnp.where(t < n_tiles - 1, post_ref[:trail, :, :], 0)
    xc = jnp.concatenate([pre, x_ref[...], post], axis=0)
    taps = [
        jax.lax.slice_in_dim(xc, k, k + l_blk, axis=0)
        for k in range(kernel_size)
    ]
    xs = jnp.concatenate(taps, axis=2)
    xs = xs.reshape(l_blk * n_total, kernel_size * c_in)
    return jax.lax.dot_general(
        xs, w_ref[...],
        dimension_numbers=(((1,), (0,)), ((), ())),
        preferred_element_type=jnp.float32)


def _stats_kernel(pre_ref, x_ref, post_ref, w_ref, stats_ref, *, kernel_size,
                  pad, l_blk, n_total, c_in, l_out, n_tiles):
    t = pl.program_id(0)
    y = _conv_tile(pre_ref, x_ref, post_ref, w_ref, kernel_size, pad, l_blk,
                   n_total, c_in, n_tiles)
    s1 = jnp.sum(y, axis=0, keepdims=True)
    s2 = jnp.sum(y * y, axis=0, keepdims=True)
    n_ragged = n_tiles * l_blk - l_out
    if n_ragged:
        yr = y.reshape(l_blk, n_total, -1)[l_blk - n_ragged:]
        yr = yr.reshape(n_ragged * n_total, -1)
        r1 = jnp.sum(yr, axis=0, keepdims=True)
        r2 = jnp.sum(yr * yr, axis=0, keepdims=True)
        last = (t == n_tiles - 1).astype(jnp.float32)
        s1 = s1 - last * r1
        s2 = s2 - last * r2
    stats_ref[...] = jnp.concatenate([s1, s2], axis=0)


def _apply_kernel(pre_ref, x_ref, post_ref, w_ref, stats_ref, g_ref, b_ref,
                  out_ref, *, kernel_size, pad, l_blk, n_total, c_in, cnt,
                  eps, n_tiles):
    st = jnp.sum(stats_ref[...], axis=0)
    inv_cnt = jnp.float32(1.0 / cnt)
    mean = st[0:1, :] * inv_cnt
    var = jnp.maximum(st[1:2, :] * inv_cnt - mean * mean, 0.0)
    scale = g_ref[...] * jax.lax.rsqrt(var + eps)
    shift = b_ref[...] - mean * scale
    y = _conv_tile(pre_ref, x_ref, post_ref, w_ref, kernel_size, pad, l_blk,
                   n_total, c_in, n_tiles)
    y = jnp.maximum(y * scale + shift, 0.0)
    out_ref[...] = y.reshape(l_blk, n_total, -1)


def kernel(x, weight, bias, gamma, beta):
    del bias
    kernel_size = weight.shape[2]
    dilation = 1
    eps = 1e-3

    n, c_in, length = x.shape
    c_out = weight.shape[0]
    pad = (dilation * (kernel_size - 1)) // 2
    halo = dilation * (kernel_size - 1)
    l_out = length + 2 * pad - halo
    assert halo < _HALO_BLK + pad and pad < _HALO_BLK
    assert n % 8 == 0 and length % _L_BLK == 0 and length % _L_BLK1 == 0

    total_units = length // _HALO_BLK

    def specs(l_blk):
        units = l_blk // _HALO_BLK
        pre = pl.BlockSpec(
            (_HALO_BLK, n, c_in),
            lambda t: (jnp.maximum(t * units - 1, 0), 0, 0))
        cur = pl.BlockSpec((l_blk, n, c_in), lambda t: (t, 0, 0))
        post = pl.BlockSpec(
            (_HALO_BLK, n, c_in),
            lambda t: (jnp.minimum(t * units + units, total_units - 1), 0, 0))
        return pre, cur, post

    xt = jnp.transpose(x, (2, 0, 1)).astype(jnp.bfloat16)
    w_t = jnp.transpose(weight, (2, 1, 0)).reshape(
        kernel_size * c_in, c_out).astype(jnp.bfloat16)
    g2 = gamma.astype(jnp.float32).reshape(1, c_out)
    b2 = beta.astype(jnp.float32).reshape(1, c_out)

    w_spec = pl.BlockSpec((kernel_size * c_in, c_out), lambda t: (0, 0))
    vmem_limit = 100 * 1024 * 1024

    n_tiles1 = _cdiv(l_out, _L_BLK1)
    pre1, cur1, post1 = specs(_L_BLK1)
    stats_parts = pl.pallas_call(
        functools.partial(_stats_kernel, kernel_size=kernel_size, pad=pad,
                          l_blk=_L_BLK1, n_total=n, c_in=c_in, l_out=l_out,
                          n_tiles=n_tiles1),
        out_shape=jax.ShapeDtypeStruct((n_tiles1, 2, c_out), jnp.float32),
        grid=(n_tiles1,),
        in_specs=[pre1, cur1, post1, w_spec],
        out_specs=pl.BlockSpec((None, 2, c_out), lambda t: (t, 0, 0)),
        compiler_params=pltpu.CompilerParams(
            dimension_semantics=("parallel",),
            vmem_limit_bytes=vmem_limit),
    )(xt, xt, xt, w_t)

    n_tiles = _cdiv(l_out, _L_BLK)
    pre2, cur2, post2 = specs(_L_BLK)
    out_t = pl.pallas_call(
        functools.partial(_apply_kernel, kernel_size=kernel_size, pad=pad,
                          l_blk=_L_BLK, n_total=n, c_in=c_in,
                          cnt=float(n * l_out), eps=eps, n_tiles=n_tiles),
        out_shape=jax.ShapeDtypeStruct((l_out, n, c_out), jnp.float32),
        grid=(n_tiles,),
        in_specs=[pre2, cur2, post2, w_spec,
                  pl.BlockSpec((n_tiles1, 2, c_out), lambda t: (0, 0, 0)),
                  pl.BlockSpec((1, c_out), lambda t: (0, 0)),
                  pl.BlockSpec((1, c_out), lambda t: (0, 0))],
        out_specs=pl.BlockSpec((_L_BLK, n, c_out), lambda t: (t, 0, 0)),
        compiler_params=pltpu.CompilerParams(
            dimension_semantics=("parallel",),
            vmem_limit_bytes=vmem_limit),
    )(xt, xt, xt, w_t, stats_parts, g2, b2)

    return jnp.transpose(out_t, (1, 2, 0))
